# baseline (speedup 1.0000x reference)
import jax
import jax.numpy as jnp
from jax import lax
from jax.experimental import pallas as pl
from jax.experimental.pallas import tpu as pltpu
from jax.experimental.pallas import tpu_sc as plsc

EMBED = 64
BATCH = 16384
NROWS = 100000
NC = 2
NS = 16
L = 16
NW = NC * NS
BPW = BATCH // NW
W = 512
LASTW = NROWS - 195 * W
HCAP = 1616
CCAP = 208
GCAP = CCAP // L
SENT = BATCH * EMBED
GBYTES = L * EMBED


def _extract_side(tblT, idx_hbm, stage_hbm, uvals, hits, ccols, cdest,
                  bufs, tailbuf, stags, drainbuf, insems, outsems):
    s = lax.axis_index("s")
    lanes = lax.iota(jnp.int32, L)
    rows_e = [e * L + lanes for e in range(EMBED // L)]

    def blk_off(k):
        return (s + 16 * k) * W

    for b in range(2):
        pltpu.async_copy(tblT.at[:, pl.ds(blk_off(b), W)], bufs[b], insems[b])

    pltpu.sync_copy(idx_hbm, uvals)

    def filt(i, cnt):
        uv = uvals[pl.ds(i * L, L)]
        m = ((uv >> 9) & 15) == s
        key = ((uv >> 13) << 23) | ((uv & 511) << 14) | (i * L + lanes)
        plsc.store_compressed(hits.at[pl.ds(cnt, L)], key, mask=m)
        return cnt + plsc.all_reduce_population_count(m)[0]

    cnt = lax.fori_loop(0, BATCH // L, filt, jnp.int32(0))
    full_mask = lanes >= 0
    plsc.store_compressed(hits.at[pl.ds(cnt, L)],
                          jnp.full((L,), 15 << 23, jnp.int32), mask=full_mask)
    nhv = (cnt + L - 1) >> 4

    def drain_n(b, n):
        def one(g, carry):
            pltpu.make_async_copy(
                stage_hbm.at[pl.ds(0, GBYTES)], drainbuf, outsems[b]).wait()
            return carry
        lax.fori_loop(0, n, one, 0)

    def process_block(kidx, b, prev_groups, buf, colhi=None, colshift=0,
                      gate=None):
        def subf(i, cc):
            key = hits[pl.ds(i * L, L)]
            m = (key >> 23) == kidx
            col = (key >> 14) & 511
            if colhi is not None:
                m = m & (col >= colshift) & (col < colhi)
            if gate is not None:
                m = m & gate
            plsc.store_compressed(
                ccols.at[pl.ds(cc, L)], col - colshift, mask=m)
            plsc.store_compressed(
                cdest.at[pl.ds(cc, L)], (key & 16383) << 6, mask=m)
            return cc + plsc.all_reduce_population_count(m)[0]

        cc = lax.fori_loop(0, nhv, subf, jnp.int32(0))
        plsc.store_compressed(ccols.at[pl.ds(cc, L)],
                              jnp.zeros((L,), jnp.int32), mask=full_mask)
        plsc.store_compressed(cdest.at[pl.ds(cc, L)],
                              jnp.full((L,), SENT, jnp.int32), mask=full_mask)

        drain_n(b, prev_groups)

        def group(g, carry):
            colv = ccols[pl.ds(g * L, L)]
            destv = cdest[pl.ds(g * L, L)]
            for l in range(L):
                col = lax.broadcast(colv[l], (L,))
                sbase = g * GBYTES + l * EMBED
                for e in range(EMBED // L):
                    stags[b][pl.ds(sbase + e * L, L)] = plsc.load_gather(
                        buf, [rows_e[e], col])
                dst = pl.multiple_of(destv[l], EMBED)
                pltpu.async_copy(stags[b].at[pl.ds(sbase, EMBED)],
                                 stage_hbm.at[pl.ds(dst, EMBED)],
                                 outsems[b])
            return carry

        ngroups = (cc + L - 1) >> 4
        lax.fori_loop(0, ngroups, group, 0)
        return ngroups

    def pair(p, carry):
        g0, g1 = carry
        news = []
        for b in range(2):
            k = 2 * p + b
            pltpu.make_async_copy(
                tblT.at[:, pl.ds(blk_off(k), W)], bufs[b], insems[b]).wait()
            news.append(process_block(k, b, (g0, g1)[b], bufs[b]))

            @pl.when(k <= 9)
            def _():
                pltpu.async_copy(tblT.at[:, pl.ds(blk_off(k + 2), W)],
                                 bufs[b], insems[b])
        return news[0], news[1]

    g0, g1 = lax.fori_loop(0, 6, pair, (jnp.int32(0), jnp.int32(0)))

    @pl.when(s <= 2)
    def _():
        pltpu.async_copy(
            tblT.at[:, pl.ds(blk_off(12), W)], bufs[0], insems[0])
        pltpu.make_async_copy(
            tblT.at[:, pl.ds(blk_off(12), W)], bufs[0], insems[0]).wait()

    @pl.when(s == 3)
    def _():
        pltpu.async_copy(tblT.at[:, pl.ds(195 * W, 128)],
                         bufs[0].at[:, pl.ds(0, 128)], insems[0])
        pltpu.async_copy(tblT.at[:, pl.ds(195 * W + 128, 32)],
                         tailbuf, insems[1])
        pltpu.make_async_copy(tblT.at[:, pl.ds(195 * W, 128)],
                              bufs[0].at[:, pl.ds(0, 128)], insems[0]).wait()
        pltpu.make_async_copy(tblT.at[:, pl.ds(195 * W + 128, 32)],
                              tailbuf, insems[1]).wait()

    is_s3 = lax.broadcast(s == 3, (L,))
    colhi_a = jnp.where(is_s3, 128, W)
    ga = process_block(12, 0, g0, bufs[0], colhi=colhi_a)
    gb = process_block(12, 1, g1, tailbuf, colhi=160, colshift=128,
                       gate=is_s3)
    drain_n(0, ga)
    drain_n(1, gb)


def _extract_body(u_hbm, v_hbm, utT, itT, ustage, vstage, uvals, hits,
                  ccols, cdest, buf0, buf1, tailbuf, stag0, stag1, drainbuf,
                  insem0, insem1, outsem0, outsem1):
    c = lax.axis_index("c")

    @pl.when(c == 0)
    def _():
        _extract_side(utT, u_hbm, ustage, uvals, hits, ccols, cdest,
                      (buf0, buf1), tailbuf, (stag0, stag1), drainbuf,
                      (insem0, insem1), (outsem0, outsem1))

    @pl.when(c == 1)
    def _():
        _extract_side(itT, v_hbm, vstage, uvals, hits, ccols, cdest,
                      (buf0, buf1), tailbuf, (stag0, stag1), drainbuf,
                      (insem0, insem1), (outsem0, outsem1))


def _dot_body(ustage, vstage, out_hbm, ubuf, vbuf, outv, sem):
    wid = lax.axis_index("s") * NC + lax.axis_index("c")
    base = wid * BPW
    lanes = lax.iota(jnp.int32, L)

    cu = pltpu.async_copy(ustage.at[pl.ds(base * EMBED, BPW * EMBED)], ubuf, sem)
    cv = pltpu.async_copy(vstage.at[pl.ds(base * EMBED, BPW * EMBED)], vbuf, sem)
    cu.wait()
    cv.wait()

    def group(g, carry):
        tot = jnp.zeros((L,), jnp.float32)
        for r in range(L):
            j = (g * L + r) * EMBED
            acc = ubuf[pl.ds(j, L)] * vbuf[pl.ds(j, L)]
            for e in range(1, EMBED // L):
                acc = acc + ubuf[pl.ds(j + e * L, L)] * vbuf[pl.ds(j + e * L, L)]
            tot = jnp.where(lanes == r, jnp.sum(acc), tot)
        outv[pl.ds(g * L, L)] = tot
        return carry

    lax.fori_loop(0, BPW // L, group, 0)
    pltpu.sync_copy(outv, out_hbm.at[pl.ds(base, BPW)])


def kernel(u, v, user_table, item_table):
    u32 = u.astype(jnp.int32)
    v32 = v.astype(jnp.int32)
    utT = user_table.T
    itT = item_table.T
    mesh = plsc.VectorSubcoreMesh(core_axis_name="c", subcore_axis_name="s")
    params = pltpu.CompilerParams(
        needs_layout_passes=False, use_tc_tiling_on_sc=True)

    extract = pl.kernel(
        _extract_body,
        out_type=(
            jax.ShapeDtypeStruct(((BATCH + 1) * EMBED,), jnp.float32),
            jax.ShapeDtypeStruct(((BATCH + 1) * EMBED,), jnp.float32),
        ),
        mesh=mesh,
        compiler_params=params,
        scratch_types=[
            pltpu.VMEM((BATCH,), jnp.int32),
            pltpu.VMEM((HCAP,), jnp.int32),
            pltpu.VMEM((CCAP,), jnp.int32),
            pltpu.VMEM((CCAP,), jnp.int32),
            pltpu.VMEM((EMBED, W), jnp.float32),
            pltpu.VMEM((EMBED, W), jnp.float32),
            pltpu.VMEM((EMBED, 32), jnp.float32),
            pltpu.VMEM((GCAP * GBYTES,), jnp.float32),
            pltpu.VMEM((GCAP * GBYTES,), jnp.float32),
            pltpu.VMEM((GBYTES,), jnp.float32),
            pltpu.SemaphoreType.DMA,
            pltpu.SemaphoreType.DMA,
            pltpu.SemaphoreType.DMA,
            pltpu.SemaphoreType.DMA,
        ],
    )
    ustage, vstage = extract(u32, v32, utT, itT)

    dot = pl.kernel(
        _dot_body,
        out_type=jax.ShapeDtypeStruct((BATCH,), jnp.float32),
        mesh=mesh,
        compiler_params=params,
        scratch_types=[
            pltpu.VMEM((BPW * EMBED,), jnp.float32),
            pltpu.VMEM((BPW * EMBED,), jnp.float32),
            pltpu.VMEM((BPW,), jnp.float32),
            pltpu.SemaphoreType.DMA,
        ],
    )
    return dot(ustage, vstage)


# speedup vs baseline: 1.0233x; 1.0233x over previous
import jax
import jax.numpy as jnp
from jax import lax
from jax.experimental import pallas as pl
from jax.experimental.pallas import tpu as pltpu
from jax.experimental.pallas import tpu_sc as plsc

EMBED = 64
BATCH = 16384
NROWS = 100000
NC = 2
NS = 16
L = 16
NW = NC * NS
BPW = BATCH // NW
W = 512
LASTW = NROWS - 195 * W
HCAP = 1616
CCAP = 208
GCAP = CCAP // L
SENT = BATCH * EMBED
GBYTES = L * EMBED


def _extract_side(tblT, idx_hbm, stage_hbm, uvals, hits, ccols, cdest,
                  bufs, tailbuf, stags, tmp, drainbuf, insems, outsems):
    s = lax.axis_index("s")
    lanes = lax.iota(jnp.int32, L)
    rows_e = [e * L + lanes for e in range(EMBED // L)]

    def blk_off(k):
        return (s + 16 * k) * W

    for b in range(2):
        pltpu.async_copy(tblT.at[:, pl.ds(blk_off(b), W)], bufs[b], insems[b])

    pltpu.sync_copy(idx_hbm, uvals)

    def filt(i, cnt):
        uv = uvals[pl.ds(i * L, L)]
        m = ((uv >> 9) & 15) == s
        key = ((uv >> 13) << 23) | ((uv & 511) << 14) | (i * L + lanes)
        plsc.store_compressed(hits.at[pl.ds(cnt, L)], key, mask=m)
        return cnt + plsc.all_reduce_population_count(m)[0]

    cnt = lax.fori_loop(0, BATCH // L, filt, jnp.int32(0))
    full_mask = lanes >= 0
    plsc.store_compressed(hits.at[pl.ds(cnt, L)],
                          jnp.full((L,), 15 << 23, jnp.int32), mask=full_mask)
    nhv = (cnt + L - 1) >> 4

    def drain_n(b, n):
        def one(g, carry):
            pltpu.make_async_copy(
                stage_hbm.at[pl.ds(0, GBYTES)], drainbuf, outsems[b]).wait()
            return carry
        lax.fori_loop(0, n, one, 0)

    def process_block(kidx, b, prev_groups, buf, colhi=None, colshift=0,
                      gate=None):
        def subf(i, cc):
            key = hits[pl.ds(i * L, L)]
            m = (key >> 23) == kidx
            col = (key >> 14) & 511
            if colhi is not None:
                m = m & (col >= colshift) & (col < colhi)
            if gate is not None:
                m = m & gate
            plsc.store_compressed(
                ccols.at[pl.ds(cc, L)], col - colshift, mask=m)
            plsc.store_compressed(
                cdest.at[pl.ds(cc, L)], (key & 16383) << 6, mask=m)
            return cc + plsc.all_reduce_population_count(m)[0]

        cc = lax.fori_loop(0, nhv, subf, jnp.int32(0))
        plsc.store_compressed(ccols.at[pl.ds(cc, L)],
                              jnp.zeros((L,), jnp.int32), mask=full_mask)
        plsc.store_compressed(cdest.at[pl.ds(cc, L)],
                              jnp.full((L,), SENT, jnp.int32), mask=full_mask)

        drain_n(b, prev_groups)

        def group(g, carry):
            colv = ccols[pl.ds(g * L, L)]
            destv = cdest[pl.ds(g * L, L)]
            for e in range(EMBED):
                row = jnp.full((L,), e, jnp.int32)
                vals = plsc.load_gather(buf, [row, colv])
                plsc.store_compressed(tmp.at[pl.ds(e * 17, L)], vals,
                                      mask=full_mask)
            for l in range(L):
                sbase = g * GBYTES + l * EMBED
                for e in range(EMBED // L):
                    stags[b][pl.ds(sbase + e * L, L)] = plsc.load_gather(
                        tmp, [(rows_e[e]) * 17 + l])
                dst = pl.multiple_of(destv[l], EMBED)
                pltpu.async_copy(stags[b].at[pl.ds(sbase, EMBED)],
                                 stage_hbm.at[pl.ds(dst, EMBED)],
                                 outsems[b])
            return carry

        ngroups = (cc + L - 1) >> 4
        lax.fori_loop(0, ngroups, group, 0)
        return ngroups

    def pair(p, carry):
        g0, g1 = carry
        news = []
        for b in range(2):
            k = 2 * p + b
            pltpu.make_async_copy(
                tblT.at[:, pl.ds(blk_off(k), W)], bufs[b], insems[b]).wait()
            news.append(process_block(k, b, (g0, g1)[b], bufs[b]))

            @pl.when(k <= 9)
            def _():
                pltpu.async_copy(tblT.at[:, pl.ds(blk_off(k + 2), W)],
                                 bufs[b], insems[b])
        return news[0], news[1]

    g0, g1 = lax.fori_loop(0, 6, pair, (jnp.int32(0), jnp.int32(0)))

    @pl.when(s <= 2)
    def _():
        pltpu.async_copy(
            tblT.at[:, pl.ds(blk_off(12), W)], bufs[0], insems[0])
        pltpu.make_async_copy(
            tblT.at[:, pl.ds(blk_off(12), W)], bufs[0], insems[0]).wait()

    @pl.when(s == 3)
    def _():
        pltpu.async_copy(tblT.at[:, pl.ds(195 * W, 128)],
                         bufs[0].at[:, pl.ds(0, 128)], insems[0])
        pltpu.async_copy(tblT.at[:, pl.ds(195 * W + 128, 32)],
                         tailbuf, insems[1])
        pltpu.make_async_copy(tblT.at[:, pl.ds(195 * W, 128)],
                              bufs[0].at[:, pl.ds(0, 128)], insems[0]).wait()
        pltpu.make_async_copy(tblT.at[:, pl.ds(195 * W + 128, 32)],
                              tailbuf, insems[1]).wait()

    is_s3 = lax.broadcast(s == 3, (L,))
    colhi_a = jnp.where(is_s3, 128, W)
    ga = process_block(12, 0, g0, bufs[0], colhi=colhi_a)
    gb = process_block(12, 1, g1, tailbuf, colhi=160, colshift=128,
                       gate=is_s3)
    drain_n(0, ga)
    drain_n(1, gb)


def _extract_body(u_hbm, v_hbm, utT, itT, ustage, vstage, uvals, hits,
                  ccols, cdest, buf0, buf1, tailbuf, stag0, stag1, tmp,
                  drainbuf, insem0, insem1, outsem0, outsem1):
    c = lax.axis_index("c")

    @pl.when(c == 0)
    def _():
        _extract_side(utT, u_hbm, ustage, uvals, hits, ccols, cdest,
                      (buf0, buf1), tailbuf, (stag0, stag1), tmp, drainbuf,
                      (insem0, insem1), (outsem0, outsem1))

    @pl.when(c == 1)
    def _():
        _extract_side(itT, v_hbm, vstage, uvals, hits, ccols, cdest,
                      (buf0, buf1), tailbuf, (stag0, stag1), tmp, drainbuf,
                      (insem0, insem1), (outsem0, outsem1))


def _dot_body(ustage, vstage, out_hbm, ubuf, vbuf, outv, sem):
    wid = lax.axis_index("s") * NC + lax.axis_index("c")
    base = wid * BPW
    lanes = lax.iota(jnp.int32, L)

    cu = pltpu.async_copy(ustage.at[pl.ds(base * EMBED, BPW * EMBED)], ubuf, sem)
    cv = pltpu.async_copy(vstage.at[pl.ds(base * EMBED, BPW * EMBED)], vbuf, sem)
    cu.wait()
    cv.wait()

    def group(g, carry):
        tot = jnp.zeros((L,), jnp.float32)
        for r in range(L):
            j = (g * L + r) * EMBED
            acc = ubuf[pl.ds(j, L)] * vbuf[pl.ds(j, L)]
            for e in range(1, EMBED // L):
                acc = acc + ubuf[pl.ds(j + e * L, L)] * vbuf[pl.ds(j + e * L, L)]
            tot = jnp.where(lanes == r, jnp.sum(acc), tot)
        outv[pl.ds(g * L, L)] = tot
        return carry

    lax.fori_loop(0, BPW // L, group, 0)
    pltpu.sync_copy(outv, out_hbm.at[pl.ds(base, BPW)])


def kernel(u, v, user_table, item_table):
    u32 = u.astype(jnp.int32)
    v32 = v.astype(jnp.int32)
    utT = user_table.T
    itT = item_table.T
    mesh = plsc.VectorSubcoreMesh(core_axis_name="c", subcore_axis_name="s")
    params = pltpu.CompilerParams(
        needs_layout_passes=False, use_tc_tiling_on_sc=True)

    extract = pl.kernel(
        _extract_body,
        out_type=(
            jax.ShapeDtypeStruct(((BATCH + 1) * EMBED,), jnp.float32),
            jax.ShapeDtypeStruct(((BATCH + 1) * EMBED,), jnp.float32),
        ),
        mesh=mesh,
        compiler_params=params,
        scratch_types=[
            pltpu.VMEM((BATCH,), jnp.int32),
            pltpu.VMEM((HCAP,), jnp.int32),
            pltpu.VMEM((CCAP,), jnp.int32),
            pltpu.VMEM((CCAP,), jnp.int32),
            pltpu.VMEM((EMBED, W), jnp.float32),
            pltpu.VMEM((EMBED, W), jnp.float32),
            pltpu.VMEM((EMBED, 32), jnp.float32),
            pltpu.VMEM((GCAP * GBYTES,), jnp.float32),
            pltpu.VMEM((GCAP * GBYTES,), jnp.float32),
            pltpu.VMEM((EMBED * 17,), jnp.float32),
            pltpu.VMEM((GBYTES,), jnp.float32),
            pltpu.SemaphoreType.DMA,
            pltpu.SemaphoreType.DMA,
            pltpu.SemaphoreType.DMA,
            pltpu.SemaphoreType.DMA,
        ],
    )
    ustage, vstage = extract(u32, v32, utT, itT)

    dot = pl.kernel(
        _dot_body,
        out_type=jax.ShapeDtypeStruct((BATCH,), jnp.float32),
        mesh=mesh,
        compiler_params=params,
        scratch_types=[
            pltpu.VMEM((BPW * EMBED,), jnp.float32),
            pltpu.VMEM((BPW * EMBED,), jnp.float32),
            pltpu.VMEM((BPW,), jnp.float32),
            pltpu.SemaphoreType.DMA,
        ],
    )
    return dot(ustage, vstage)
